# baseline (speedup 1.0000x reference)
import functools

import jax
import jax.numpy as jnp
from jax.experimental import pallas as pl

BM = 2048
BN = 512


def _matmul_bias_kernel(x_ref, w_ref, b_ref, o_ref):
    acc = jax.lax.dot_general(
        x_ref[...].astype(jnp.bfloat16),
        w_ref[...].astype(jnp.bfloat16),
        dimension_numbers=(((1,), (1,)), ((), ())),
        preferred_element_type=jnp.float32,
    )
    o_ref[...] = acc + b_ref[...]


@functools.partial(jax.jit, static_argnums=())
def kernel(x, W, b):
    M, K = x.shape
    N = W.shape[0]
    b2 = b.reshape(1, N)
    grid = (M // BM, N // BN)
    return pl.pallas_call(
        _matmul_bias_kernel,
        grid=grid,
        in_specs=[
            pl.BlockSpec((BM, K), lambda i, j: (i, 0)),
            pl.BlockSpec((BN, K), lambda i, j: (j, 0)),
            pl.BlockSpec((1, BN), lambda i, j: (0, j)),
        ],
        out_specs=pl.BlockSpec((BM, BN), lambda i, j: (i, j)),
        out_shape=jax.ShapeDtypeStruct((M, N), jnp.float32),
    )(x, W, b2)


# speedup vs baseline: 1.0059x; 1.0059x over previous
import functools

import jax
import jax.numpy as jnp
from jax.experimental import pallas as pl
from jax.experimental.pallas import tpu as pltpu

BM = 2048
BN = 512


def _matmul_bias_kernel(x_ref, w_ref, b_ref, o_ref):
    acc = jax.lax.dot_general(
        x_ref[...],
        w_ref[...],
        dimension_numbers=(((1,), (1,)), ((), ())),
        preferred_element_type=jnp.float32,
    )
    o_ref[...] = acc + b_ref[...]


@functools.partial(jax.jit, static_argnums=())
def kernel(x, W, b):
    M, K = x.shape
    N = W.shape[0]
    b2 = b.reshape(1, N)
    grid = (M // BM, N // BN)
    return pl.pallas_call(
        _matmul_bias_kernel,
        grid=grid,
        in_specs=[
            pl.BlockSpec((BM, K), lambda i, j: (i, 0)),
            pl.BlockSpec((BN, K), lambda i, j: (j, 0)),
            pl.BlockSpec((1, BN), lambda i, j: (0, j)),
        ],
        out_specs=pl.BlockSpec((BM, BN), lambda i, j: (i, j)),
        out_shape=jax.ShapeDtypeStruct((M, N), jnp.float32),
        compiler_params=pltpu.CompilerParams(
            dimension_semantics=("parallel", "parallel"),
        ),
    )(x, W, b2)


# speedup vs baseline: 1.0087x; 1.0028x over previous
import functools

import jax
import jax.numpy as jnp
from jax.experimental import pallas as pl
from jax.experimental.pallas import tpu as pltpu

BN = 256


def _matmul_bias_kernel(x_hbm, w_ref, b_ref, o_ref, x_vmem, sem):
    @pl.when(pl.program_id(0) == 0)
    def _load_x():
        copy = pltpu.make_async_copy(x_hbm, x_vmem, sem)
        copy.start()
        copy.wait()

    acc = jax.lax.dot_general(
        x_vmem[...],
        w_ref[...],
        dimension_numbers=(((1,), (1,)), ((), ())),
        preferred_element_type=jnp.float32,
    )
    o_ref[...] = acc + b_ref[...]


@functools.partial(jax.jit, static_argnums=())
def kernel(x, W, b):
    M, K = x.shape
    N = W.shape[0]
    b2 = b.reshape(1, N)
    return pl.pallas_call(
        _matmul_bias_kernel,
        grid=(N // BN,),
        in_specs=[
            pl.BlockSpec(memory_space=pl.ANY),
            pl.BlockSpec((BN, K), lambda j: (j, 0)),
            pl.BlockSpec((1, BN), lambda j: (0, j)),
        ],
        out_specs=pl.BlockSpec((M, BN), lambda j: (0, j)),
        out_shape=jax.ShapeDtypeStruct((M, N), jnp.float32),
        scratch_shapes=[
            pltpu.VMEM((M, K), jnp.float32),
            pltpu.SemaphoreType.DMA,
        ],
        compiler_params=pltpu.CompilerParams(
            dimension_semantics=("arbitrary",),
        ),
    )(x, W, b2)
